# baseline (speedup 1.0000x reference)
import functools

import jax
import jax.numpy as jnp
from jax.experimental import pallas as pl
from jax.experimental.pallas import tpu as pltpu
from jax.experimental.pallas import tpu_sc as plsc

_T = 2048
_H = 768
_INTER = 1024
_E = 64
_CAP = 128
_ROWS = (_E + 1) * _CAP
_CHUNK = 256

_NC, _NS = 2, 16
_NW = _NC * _NS
_TPW = _T // _NW


def _router_body(xf_ref, wg_ref, usage_ref, lbl_ref, top_ref, dst_ref, xb_ref):
    xf = xf_ref[...]
    wg = wg_ref[...]
    xb_ref[...] = xf.astype(jnp.bfloat16)
    logits_t = jax.lax.dot_general(
        wg, xf, (((1,), (1,)), ((), ())), preferred_element_type=jnp.float32)
    m = jnp.max(logits_t, axis=0, keepdims=True)
    ex = jnp.exp(logits_t - m)
    s = jnp.sum(ex, axis=0, keepdims=True)
    probs_t = ex / s
    usage = jnp.mean(probs_t, axis=1, keepdims=True)
    usage_ref[...] = usage
    lbl_ref[...] = float(_E) * jnp.sum(usage * usage, keepdims=True)
    iota_e = jax.lax.broadcasted_iota(jnp.int32, (_E, 1), 0)
    e_t = jnp.min(jnp.where(logits_t == m, iota_e, _E), axis=0)
    top_ref[...] = e_t[None, :]

    r_i = jax.lax.broadcasted_iota(jnp.int32, (_CHUNK, _CHUNK), 0)
    c_i = jax.lax.broadcasted_iota(jnp.int32, (_CHUNK, _CHUNK), 1)
    upper = (r_i < c_i).astype(jnp.float32)
    carry = jnp.zeros((_E, 1), jnp.float32)
    for i in range(_T // _CHUNK):
        ech = jax.lax.slice(e_t, (i * _CHUNK,), ((i + 1) * _CHUNK,))
        ohc = (iota_e == ech[None, :]).astype(jnp.float32)
        posm = jax.lax.dot_general(
            ohc, upper, (((1,), (0,)), ((), ())),
            preferred_element_type=jnp.float32) + carry
        pos = jnp.sum(posm * ohc, axis=0).astype(jnp.int32)
        carry = carry + jnp.sum(ohc, axis=1, keepdims=True)
        dst_ref[0, i * _CHUNK:(i + 1) * _CHUNK] = jnp.where(
            pos < _CAP, ech * _CAP + pos, _E * _CAP)
    return


_router = pl.pallas_call(
    _router_body,
    out_shape=(
        jax.ShapeDtypeStruct((_E, 1), jnp.float32),
        jax.ShapeDtypeStruct((1, 1), jnp.float32),
        jax.ShapeDtypeStruct((1, _T), jnp.int32),
        jax.ShapeDtypeStruct((1, _T), jnp.int32),
        jax.ShapeDtypeStruct((_T, _H), jnp.bfloat16),
    ),
)


def _ffn_body(buf_ref, wup_ref, bup_ref, wdown_ref, bdown_ref, out_ref):
    e = pl.program_id(0)
    x = buf_ref[0].astype(jnp.float32)
    h = jax.lax.dot_general(
        x, wup_ref[0], (((1,), (0,)), ((), ())),
        preferred_element_type=jnp.float32) + bup_ref[0]
    h = 0.5 * h * (1.0 + jax.lax.erf(h * 0.7071067811865476))
    y = jax.lax.dot_general(
        h, wdown_ref[0], (((1,), (0,)), ((), ())),
        preferred_element_type=jnp.float32) + bdown_ref[0]
    out_ref[0] = jnp.where(e < _E, y, 0.0).astype(jnp.bfloat16)


_ffn = pl.pallas_call(
    _ffn_body,
    grid=(_E + 1,),
    in_specs=[
        pl.BlockSpec((1, _CAP, _H), lambda e: (e, 0, 0)),
        pl.BlockSpec((1, _H, _INTER), lambda e: (jnp.minimum(e, _E - 1), 0, 0)),
        pl.BlockSpec((1, 1, _INTER), lambda e: (jnp.minimum(e, _E - 1), 0, 0)),
        pl.BlockSpec((1, _INTER, _H), lambda e: (jnp.minimum(e, _E - 1), 0, 0)),
        pl.BlockSpec((1, 1, _H), lambda e: (jnp.minimum(e, _E - 1), 0, 0)),
    ],
    out_specs=pl.BlockSpec((1, _CAP, _H), lambda e: (e, 0, 0)),
    out_shape=jax.ShapeDtypeStruct((_E + 1, _CAP, _H), jnp.bfloat16),
)

_HW = _H // 2

@functools.lru_cache(maxsize=None)
def _sc_kernels():
    mesh = plsc.VectorSubcoreMesh(core_axis_name="c", subcore_axis_name="s")

    @functools.partial(
        pl.kernel, mesh=mesh,
        out_type=jax.ShapeDtypeStruct((_ROWS, _HW), jnp.int32),
        scratch_types=[
            pltpu.VMEM((_TPW,), jnp.int32),
            pltpu.VMEM((_TPW, _HW), jnp.int32),
            pltpu.SemaphoreType.DMA,
            pltpu.SemaphoreType.DMA,
        ],
    )
    def dispatch(xb_hbm, dst_hbm, buf_hbm, idx_v, rows_v, sem1, sem2):
        wid = jax.lax.axis_index("s") * _NC + jax.lax.axis_index("c")
        base = wid * _TPW
        cp1 = pltpu.async_copy(dst_hbm.at[pl.ds(base, _TPW)], idx_v, sem1)
        cp2 = pltpu.async_copy(xb_hbm.at[pl.ds(base, _TPW)], rows_v, sem2)
        cp1.wait()
        cp2.wait()
        pltpu.async_copy(rows_v, buf_hbm.at[idx_v], sem1).wait()

    @functools.partial(
        pl.kernel, mesh=mesh,
        out_type=jax.ShapeDtypeStruct((_T, _HW), jnp.int32),
        scratch_types=[
            pltpu.VMEM((_TPW,), jnp.int32),
            pltpu.VMEM((_TPW, _HW), jnp.int32),
            pltpu.SemaphoreType.DMA,
        ],
    )
    def combine(eo_hbm, dst_hbm, out_hbm, idx_v, rows_v, sem):
        wid = jax.lax.axis_index("s") * _NC + jax.lax.axis_index("c")
        base = wid * _TPW
        pltpu.sync_copy(dst_hbm.at[pl.ds(base, _TPW)], idx_v)
        pltpu.async_copy(eo_hbm.at[idx_v], rows_v, sem).wait()
        pltpu.sync_copy(rows_v, out_hbm.at[pl.ds(base, _TPW)])

    return dispatch, combine


def kernel(x, Wg, Wup, bup, Wdown, bdown):
    b, s, h = x.shape
    xf = x.reshape(b * s, h)
    dispatch, combine = _sc_kernels()
    usage, lbl, top2d, dst2d, xb = _router(xf, Wg)
    dst = dst2d.reshape(b * s)
    xb_i32 = jax.lax.bitcast_convert_type(
        xb.reshape(b * s, _HW, 2), jnp.int32)
    buf = dispatch(xb_i32, dst)
    buf_bf = jax.lax.bitcast_convert_type(
        buf, jnp.bfloat16).reshape(_E + 1, _CAP, h)
    eo = _ffn(buf_bf, Wup,
              bup.reshape(_E, 1, _INTER), Wdown, bdown.reshape(_E, 1, h))
    eo_i32 = jax.lax.bitcast_convert_type(
        eo.reshape(_ROWS, _HW, 2), jnp.int32)
    out = combine(eo_i32, dst)
    out_f = jax.lax.bitcast_convert_type(
        out, jnp.bfloat16).reshape(b * s, h).astype(jnp.float32)
    return (out_f.reshape(b, s, h), lbl[0, 0], usage.reshape(_E),
            top2d.reshape(b, s))


# speedup vs baseline: 2.8981x; 2.8981x over previous
import functools

import jax
import jax.numpy as jnp
from jax.experimental import pallas as pl
from jax.experimental.pallas import tpu as pltpu
from jax.experimental.pallas import tpu_sc as plsc

_T = 2048
_H = 768
_INTER = 1024
_E = 64
_CAP = 128
_ROWS = (_E + 1) * _CAP
_CHUNK = 256

_NC, _NS = 2, 16
_NW = _NC * _NS
_TPW = _T // _NW


_HW = _H // 2


def _pack_rows(y):
    yb = y.astype(jnp.bfloat16)
    lo = jax.lax.bitcast_convert_type(yb[:, :_HW], jnp.uint16).astype(jnp.uint32)
    hi = jax.lax.bitcast_convert_type(yb[:, _HW:], jnp.uint16).astype(jnp.uint32)
    return jax.lax.bitcast_convert_type(lo | (hi << 16), jnp.int32)


def _unpack_rows(v):
    vu = jax.lax.bitcast_convert_type(v, jnp.uint32)
    lo = jax.lax.bitcast_convert_type((vu & 0xFFFF).astype(jnp.uint16),
                                      jnp.bfloat16)
    hi = jax.lax.bitcast_convert_type((vu >> 16).astype(jnp.uint16),
                                      jnp.bfloat16)
    return jnp.concatenate([lo, hi], axis=1)


def _router_body(xf_ref, wg_ref, usage_ref, lbl_ref, top_ref, dst_ref, xb_ref):
    xf = xf_ref[...]
    wg = wg_ref[...]
    xb_ref[...] = _pack_rows(xf)
    logits_t = jax.lax.dot_general(
        wg, xf, (((1,), (1,)), ((), ())), preferred_element_type=jnp.float32)
    m = jnp.max(logits_t, axis=0, keepdims=True)
    ex = jnp.exp(logits_t - m)
    s = jnp.sum(ex, axis=0, keepdims=True)
    probs_t = ex / s
    usage = jnp.mean(probs_t, axis=1, keepdims=True)
    usage_ref[...] = usage
    lbl_ref[...] = float(_E) * jnp.sum(usage * usage, keepdims=True)
    iota_e = jax.lax.broadcasted_iota(jnp.int32, (_E, 1), 0)
    e_t = jnp.min(jnp.where(logits_t == m, iota_e, _E), axis=0)
    top_ref[...] = e_t[None, :]

    r_i = jax.lax.broadcasted_iota(jnp.int32, (_CHUNK, _CHUNK), 0)
    c_i = jax.lax.broadcasted_iota(jnp.int32, (_CHUNK, _CHUNK), 1)
    upper = (r_i < c_i).astype(jnp.float32)
    carry = jnp.zeros((_E, 1), jnp.float32)
    for i in range(_T // _CHUNK):
        ech = jax.lax.slice(e_t, (i * _CHUNK,), ((i + 1) * _CHUNK,))
        ohc = (iota_e == ech[None, :]).astype(jnp.float32)
        posm = jax.lax.dot_general(
            ohc, upper, (((1,), (0,)), ((), ())),
            preferred_element_type=jnp.float32) + carry
        pos = jnp.sum(posm * ohc, axis=0).astype(jnp.int32)
        carry = carry + jnp.sum(ohc, axis=1, keepdims=True)
        dst_ref[0, i * _CHUNK:(i + 1) * _CHUNK] = jnp.where(
            pos < _CAP, ech * _CAP + pos, _E * _CAP)
    return


_router = pl.pallas_call(
    _router_body,
    out_shape=(
        jax.ShapeDtypeStruct((_E, 1), jnp.float32),
        jax.ShapeDtypeStruct((1, 1), jnp.float32),
        jax.ShapeDtypeStruct((1, _T), jnp.int32),
        jax.ShapeDtypeStruct((1, _T), jnp.int32),
        jax.ShapeDtypeStruct((_T, _HW), jnp.int32),
    ),
)


def _ffn_body(buf_ref, wup_ref, bup_ref, wdown_ref, bdown_ref, out_ref):
    e = pl.program_id(0)
    x = _unpack_rows(buf_ref[0]).astype(jnp.float32)
    h = jax.lax.dot_general(
        x, wup_ref[0], (((1,), (0,)), ((), ())),
        preferred_element_type=jnp.float32) + bup_ref[0]
    h = 0.5 * h * (1.0 + jax.lax.erf(h * 0.7071067811865476))
    y = jax.lax.dot_general(
        h, wdown_ref[0], (((1,), (0,)), ((), ())),
        preferred_element_type=jnp.float32) + bdown_ref[0]
    out_ref[0] = _pack_rows(jnp.where(e < _E, y, 0.0))


_ffn = pl.pallas_call(
    _ffn_body,
    grid=(_E + 1,),
    in_specs=[
        pl.BlockSpec((1, _CAP, _HW), lambda e: (e, 0, 0)),
        pl.BlockSpec((1, _H, _INTER), lambda e: (jnp.minimum(e, _E - 1), 0, 0)),
        pl.BlockSpec((1, 1, _INTER), lambda e: (jnp.minimum(e, _E - 1), 0, 0)),
        pl.BlockSpec((1, _INTER, _H), lambda e: (jnp.minimum(e, _E - 1), 0, 0)),
        pl.BlockSpec((1, 1, _H), lambda e: (jnp.minimum(e, _E - 1), 0, 0)),
    ],
    out_specs=pl.BlockSpec((1, _CAP, _HW), lambda e: (e, 0, 0)),
    out_shape=jax.ShapeDtypeStruct((_E + 1, _CAP, _HW), jnp.int32),
)

@functools.lru_cache(maxsize=None)
def _sc_kernels():
    mesh = plsc.VectorSubcoreMesh(core_axis_name="c", subcore_axis_name="s")

    @functools.partial(
        pl.kernel, mesh=mesh,
        out_type=jax.ShapeDtypeStruct((_ROWS, _HW), jnp.int32),
        scratch_types=[
            pltpu.VMEM((_TPW,), jnp.int32),
            pltpu.VMEM((_TPW, _HW), jnp.int32),
            pltpu.SemaphoreType.DMA,
            pltpu.SemaphoreType.DMA,
        ],
    )
    def dispatch(xb_hbm, dst_hbm, buf_hbm, idx_v, rows_v, sem1, sem2):
        wid = jax.lax.axis_index("s") * _NC + jax.lax.axis_index("c")
        base = wid * _TPW
        cp1 = pltpu.async_copy(dst_hbm.at[pl.ds(base, _TPW)], idx_v, sem1)
        cp2 = pltpu.async_copy(xb_hbm.at[pl.ds(base, _TPW)], rows_v, sem2)
        cp1.wait()
        cp2.wait()
        pltpu.async_copy(rows_v, buf_hbm.at[idx_v], sem1).wait()

    @functools.partial(
        pl.kernel, mesh=mesh,
        out_type=jax.ShapeDtypeStruct((_T, _HW), jnp.int32),
        scratch_types=[
            pltpu.VMEM((_TPW,), jnp.int32),
            pltpu.VMEM((_TPW, _HW), jnp.int32),
            pltpu.SemaphoreType.DMA,
        ],
    )
    def combine(eo_hbm, dst_hbm, out_hbm, idx_v, rows_v, sem):
        wid = jax.lax.axis_index("s") * _NC + jax.lax.axis_index("c")
        base = wid * _TPW
        pltpu.sync_copy(dst_hbm.at[pl.ds(base, _TPW)], idx_v)
        pltpu.async_copy(eo_hbm.at[idx_v], rows_v, sem).wait()
        pltpu.sync_copy(rows_v, out_hbm.at[pl.ds(base, _TPW)])

    return dispatch, combine


def kernel(x, Wg, Wup, bup, Wdown, bdown):
    b, s, h = x.shape
    xf = x.reshape(b * s, h)
    dispatch, combine = _sc_kernels()
    usage, lbl, top2d, dst2d, xb = _router(xf, Wg)
    dst = dst2d.reshape(b * s)
    buf = dispatch(xb, dst)
    eo = _ffn(buf.reshape(_E + 1, _CAP, _HW), Wup,
              bup.reshape(_E, 1, _INTER), Wdown, bdown.reshape(_E, 1, h))
    out = combine(eo.reshape(_ROWS, _HW), dst)
    out_f = _unpack_rows(out).astype(jnp.float32)
    return (out_f.reshape(b, s, h), lbl[0, 0], usage.reshape(_E),
            top2d.reshape(b, s))


# speedup vs baseline: 2.9022x; 1.0014x over previous
import functools

import jax
import jax.numpy as jnp
from jax.experimental import pallas as pl
from jax.experimental.pallas import tpu as pltpu
from jax.experimental.pallas import tpu_sc as plsc

_T = 2048
_H = 768
_INTER = 1024
_E = 64
_CAP = 128
_ROWS = (_E + 1) * _CAP
_CHUNK = 256

_NC, _NS = 2, 16
_NW = _NC * _NS
_TPW = _T // _NW


_HW = _H // 2


def _pack_rows(y):
    yb = y.astype(jnp.bfloat16)
    lo = jax.lax.bitcast_convert_type(yb[:, :_HW], jnp.uint16).astype(jnp.uint32)
    hi = jax.lax.bitcast_convert_type(yb[:, _HW:], jnp.uint16).astype(jnp.uint32)
    return jax.lax.bitcast_convert_type(lo | (hi << 16), jnp.int32)


def _unpack_rows(v):
    vu = jax.lax.bitcast_convert_type(v, jnp.uint32)
    lo = jax.lax.bitcast_convert_type((vu & 0xFFFF).astype(jnp.uint16),
                                      jnp.bfloat16)
    hi = jax.lax.bitcast_convert_type((vu >> 16).astype(jnp.uint16),
                                      jnp.bfloat16)
    return jnp.concatenate([lo, hi], axis=1)


def _router_body(xf_ref, wg_ref, usage_ref, lbl_ref, top_ref, dst_ref, xb_ref):
    xf = xf_ref[...]
    wg = wg_ref[...]
    xb_ref[...] = _pack_rows(xf)
    logits_t = jax.lax.dot_general(
        wg, xf, (((1,), (1,)), ((), ())), preferred_element_type=jnp.float32)
    m = jnp.max(logits_t, axis=0, keepdims=True)
    ex = jnp.exp(logits_t - m)
    s = jnp.sum(ex, axis=0, keepdims=True)
    probs_t = ex / s
    usage = jnp.mean(probs_t, axis=1, keepdims=True)
    usage_ref[...] = usage
    lbl_ref[...] = float(_E) * jnp.sum(usage * usage, keepdims=True)
    iota_e = jax.lax.broadcasted_iota(jnp.int32, (_E, 1), 0)
    e_t = jnp.min(jnp.where(logits_t == m, iota_e, _E), axis=0)
    top_ref[...] = e_t[None, :]

    r_i = jax.lax.broadcasted_iota(jnp.int32, (_CHUNK, _CHUNK), 0)
    c_i = jax.lax.broadcasted_iota(jnp.int32, (_CHUNK, _CHUNK), 1)
    upper = (r_i < c_i).astype(jnp.float32)
    carry = jnp.zeros((_E, 1), jnp.float32)
    for i in range(_T // _CHUNK):
        ech = jax.lax.slice(e_t, (i * _CHUNK,), ((i + 1) * _CHUNK,))
        ohc = (iota_e == ech[None, :]).astype(jnp.float32)
        posm = jax.lax.dot_general(
            ohc, upper, (((1,), (0,)), ((), ())),
            preferred_element_type=jnp.float32) + carry
        pos = jnp.sum(posm * ohc, axis=0).astype(jnp.int32)
        carry = carry + jnp.sum(ohc, axis=1, keepdims=True)
        dst_ref[0, i * _CHUNK:(i + 1) * _CHUNK] = jnp.where(
            pos < _CAP, ech * _CAP + pos, _E * _CAP)
    return


_router = pl.pallas_call(
    _router_body,
    out_shape=(
        jax.ShapeDtypeStruct((_E, 1), jnp.float32),
        jax.ShapeDtypeStruct((1, 1), jnp.float32),
        jax.ShapeDtypeStruct((1, _T), jnp.int32),
        jax.ShapeDtypeStruct((1, _T), jnp.int32),
        jax.ShapeDtypeStruct((_T, _HW), jnp.int32),
    ),
)


def _ffn_body(buf_ref, wup_ref, bup_ref, wdown_ref, bdown_ref, out_ref):
    e = pl.program_id(0)
    x = _unpack_rows(buf_ref[0]).astype(jnp.float32)
    h = jax.lax.dot_general(
        x, wup_ref[0], (((1,), (0,)), ((), ())),
        preferred_element_type=jnp.float32) + bup_ref[0]
    h = 0.5 * h * (1.0 + jax.lax.erf(h * 0.7071067811865476))
    y = jax.lax.dot_general(
        h, wdown_ref[0], (((1,), (0,)), ((), ())),
        preferred_element_type=jnp.float32) + bdown_ref[0]
    out_ref[0] = _pack_rows(jnp.where(e < _E, y, 0.0))


_ffn = pl.pallas_call(
    _ffn_body,
    grid=(_E + 1,),
    in_specs=[
        pl.BlockSpec((1, _CAP, _HW), lambda e: (e, 0, 0)),
        pl.BlockSpec((1, _H, _INTER), lambda e: (jnp.minimum(e, _E - 1), 0, 0)),
        pl.BlockSpec((1, 1, _INTER), lambda e: (jnp.minimum(e, _E - 1), 0, 0)),
        pl.BlockSpec((1, _INTER, _H), lambda e: (jnp.minimum(e, _E - 1), 0, 0)),
        pl.BlockSpec((1, 1, _H), lambda e: (jnp.minimum(e, _E - 1), 0, 0)),
    ],
    out_specs=pl.BlockSpec((1, _CAP, _HW), lambda e: (e, 0, 0)),
    out_shape=jax.ShapeDtypeStruct((_E + 1, _CAP, _HW), jnp.int32),
)

@functools.lru_cache(maxsize=None)
def _sc_kernels():
    mesh = plsc.VectorSubcoreMesh(core_axis_name="c", subcore_axis_name="s")

    half = _TPW // 2
    sc_scratch = [
        pltpu.VMEM((half,), jnp.int32),
        pltpu.VMEM((half,), jnp.int32),
        pltpu.VMEM((half, _HW), jnp.int32),
        pltpu.VMEM((half, _HW), jnp.int32),
        pltpu.SemaphoreType.DMA,
        pltpu.SemaphoreType.DMA,
        pltpu.SemaphoreType.DMA,
        pltpu.SemaphoreType.DMA,
    ]

    @functools.partial(
        pl.kernel, mesh=mesh,
        out_type=jax.ShapeDtypeStruct((_ROWS, _HW), jnp.int32),
        scratch_types=sc_scratch,
    )
    def dispatch(xb_hbm, dst_hbm, buf_hbm,
                 idx_a, idx_b, rows_a, rows_b, s1, s2, s3, s4):
        wid = jax.lax.axis_index("s") * _NC + jax.lax.axis_index("c")
        base = wid * _TPW
        i0 = pltpu.async_copy(dst_hbm.at[pl.ds(base, half)], idx_a, s1)
        i1 = pltpu.async_copy(dst_hbm.at[pl.ds(base + half, half)], idx_b, s2)
        r0 = pltpu.async_copy(xb_hbm.at[pl.ds(base, half)], rows_a, s3)
        r1 = pltpu.async_copy(xb_hbm.at[pl.ds(base + half, half)], rows_b, s4)
        i0.wait()
        r0.wait()
        w0 = pltpu.async_copy(rows_a, buf_hbm.at[idx_a], s1)
        i1.wait()
        r1.wait()
        w1 = pltpu.async_copy(rows_b, buf_hbm.at[idx_b], s2)
        w0.wait()
        w1.wait()

    @functools.partial(
        pl.kernel, mesh=mesh,
        out_type=jax.ShapeDtypeStruct((_T, _HW), jnp.int32),
        scratch_types=sc_scratch,
    )
    def combine(eo_hbm, dst_hbm, out_hbm,
                idx_a, idx_b, rows_a, rows_b, s1, s2, s3, s4):
        wid = jax.lax.axis_index("s") * _NC + jax.lax.axis_index("c")
        base = wid * _TPW
        i0 = pltpu.async_copy(dst_hbm.at[pl.ds(base, half)], idx_a, s1)
        i1 = pltpu.async_copy(dst_hbm.at[pl.ds(base + half, half)], idx_b, s2)
        i0.wait()
        g0 = pltpu.async_copy(eo_hbm.at[idx_a], rows_a, s3)
        i1.wait()
        g1 = pltpu.async_copy(eo_hbm.at[idx_b], rows_b, s4)
        g0.wait()
        w0 = pltpu.async_copy(rows_a, out_hbm.at[pl.ds(base, half)], s1)
        g1.wait()
        w1 = pltpu.async_copy(rows_b, out_hbm.at[pl.ds(base + half, half)], s2)
        w0.wait()
        w1.wait()

    return dispatch, combine


def kernel(x, Wg, Wup, bup, Wdown, bdown):
    b, s, h = x.shape
    xf = x.reshape(b * s, h)
    dispatch, combine = _sc_kernels()
    usage, lbl, top2d, dst2d, xb = _router(xf, Wg)
    dst = dst2d.reshape(b * s)
    buf = dispatch(xb, dst)
    eo = _ffn(buf.reshape(_E + 1, _CAP, _HW), Wup,
              bup.reshape(_E, 1, _INTER), Wdown, bdown.reshape(_E, 1, h))
    out = combine(eo.reshape(_ROWS, _HW), dst)
    out_f = _unpack_rows(out).astype(jnp.float32)
    return (out_f.reshape(b, s, h), lbl[0, 0], usage.reshape(_E),
            top2d.reshape(b, s))
